# Initial kernel scaffold; baseline (speedup 1.0000x reference)
import jax
import jax.numpy as jnp
from jax.experimental import pallas as pl


def kernel(predictions, targets):
    raise NotImplementedError("write your pallas kernel here")


# speedup vs baseline: 2.4533x; 2.4533x over previous
import jax
import jax.numpy as jnp
import numpy as np
from jax.experimental import pallas as pl
from jax.experimental.pallas import tpu as pltpu

_OUT_SIZE = (4, 64, 64)
_REAL_SIZE = (3, 50, 50)
_SCALEF = 1.4
_THRESH = 0.5
_RAD = {"O": 0.74 * _SCALEF, "H": 0.528 * _SCALEF}
_N = _OUT_SIZE[0] * _OUT_SIZE[1] * _OUT_SIZE[2]
_ROWS, _COLS = 128, 128


def _grid_ind():
    Z, X, Y = _OUT_SIZE
    zz, xx, yy = np.meshgrid(np.arange(Z), np.arange(X), np.arange(Y), indexing="ij")
    ind = np.stack([np.zeros_like(zz), zz, xx, yy], 0).reshape(4, -1)
    return jnp.asarray(ind, dtype=jnp.float32)


def _chan_scale():
    return jnp.asarray(
        [1.0] + [r / float(o) for o, r in zip(_OUT_SIZE, _REAL_SIZE)],
        dtype=jnp.float32,
    )


def _organize(x, ind, scale):
    b = x.shape[0]
    e = x.shape[1] // 4
    Z, X, Y = _OUT_SIZE
    out = x.reshape(b, 4, e, Z * X * Y)
    out = jnp.transpose(out, (0, 2, 1, 3))
    out = out + ind[None, None]
    return out * scale[None, None, :, None]


def _slice_kernel(r2_ref,
                  cP_ref, zP_ref, xP_ref, cPb_ref, zPb_ref, xPb_ref,
                  cT_ref, zT_ref, xT_ref, cTb_ref, zTb_ref, xTb_ref,
                  out_ref, kP_ref, kT_ref, st_ref):
    r2 = r2_ref[0, 0, 0]
    rows = jax.lax.broadcasted_iota(jnp.int32, (_ROWS, _COLS), 0)
    cols = jax.lax.broadcasted_iota(jnp.int32, (_ROWS, _COLS), 1)
    flat = rows * _COLS + cols
    lane = jax.lax.broadcasted_iota(jnp.int32, (1, _COLS), 1)

    def _extract(ref, hi, lo):
        row = ref[0, pl.ds(hi, 1), :]
        oh = (lane == lo).astype(jnp.float32)
        return jnp.sum(row * oh)

    def _extract2(ref, hi, lo):
        row = ref[pl.ds(hi, 1), :]
        oh = (lane == lo).astype(jnp.float32)
        return jnp.sum(row * oh)

    def _nms(c_ref, z_ref, x_ref, cb_ref, zb_ref, xb_ref, k_ref):
        conf = c_ref[0]
        k_ref[:, :] = jnp.where(conf > _THRESH, 1.0, 0.0)

        def body(i, carry):
            hi = i // _COLS
            lo = i % _COLS
            ci = _extract(c_ref, hi, lo)
            zi = _extract(z_ref, hi, lo)
            xi = _extract(x_ref, hi, lo)
            cbi = _extract(cb_ref, hi, lo)
            zbi = _extract(zb_ref, hi, lo)
            xbi = _extract(xb_ref, hi, lo)
            cc = c_ref[0]
            zz = z_ref[0]
            xx = x_ref[0]
            ni = ci * ci + zi * zi + xi * xi
            nrm = cc * cc + zz * zz + xx * xx
            dot = cbi * cb_ref[0] + zbi * zb_ref[0] + xbi * xb_ref[0]
            d2 = ni + nrm - 2.0 * dot
            close = d2 < r2
            keep = k_ref[:, :] > 0.0
            sup = jnp.any(close & keep & (flat < i))
            rowk = k_ref[pl.ds(hi, 1), :]
            newrow = jnp.where((lane == lo) & sup, 0.0, rowk)
            k_ref[pl.ds(hi, 1), :] = newrow
            return carry

        jax.lax.fori_loop(0, _N, body, 0)

    _nms(cP_ref, zP_ref, xP_ref, cPb_ref, zPb_ref, xPb_ref, kP_ref)
    _nms(cT_ref, zT_ref, xT_ref, cTb_ref, zTb_ref, xTb_ref, kT_ref)

    st_ref[:, :] = jnp.zeros((_ROWS, _COLS), jnp.float32)

    def mbody(i, tp):
        hi = i // _COLS
        lo = i % _COLS
        ci = _extract(cP_ref, hi, lo)
        zi = _extract(zP_ref, hi, lo)
        xi = _extract(xP_ref, hi, lo)
        cbi = _extract(cPb_ref, hi, lo)
        zbi = _extract(zPb_ref, hi, lo)
        xbi = _extract(xPb_ref, hi, lo)
        kpi = _extract2(kP_ref, hi, lo)
        cc = cT_ref[0]
        zz = zT_ref[0]
        xx = xT_ref[0]
        ni = ci * ci + zi * zi + xi * xi
        nrm = cc * cc + zz * zz + xx * xx
        dot = cbi * cTb_ref[0] + zbi * zTb_ref[0] + xbi * xTb_ref[0]
        d2 = ni + nrm - 2.0 * dot
        cand = (d2 < r2) & (kT_ref[:, :] > 0.0) & (st_ref[:, :] <= 0.0) & (kpi > 0.0)
        has = jnp.any(cand)
        bb = jnp.min(jnp.where(cand, flat, _N))
        bb = jnp.where(has, bb, 0)
        bhi = bb // _COLS
        blo = bb % _COLS
        rowst = st_ref[pl.ds(bhi, 1), :]
        newrow = jnp.where((lane == blo) & has, 1.0, rowst)
        st_ref[pl.ds(bhi, 1), :] = newrow
        return tp + jnp.where(has, 1.0, 0.0)

    tp = jax.lax.fori_loop(0, _N, mbody, jnp.float32(0.0))
    fp = jnp.sum(kP_ref[:, :]) - tp
    fn = jnp.sum(kT_ref[:, :]) - tp

    r8 = jax.lax.broadcasted_iota(jnp.int32, (8, 128), 0)
    l8 = jax.lax.broadcasted_iota(jnp.int32, (8, 128), 1)
    vals = (jnp.where((r8 == 0) & (l8 == 0), tp, 0.0)
            + jnp.where((r8 == 0) & (l8 == 1), fp, 0.0)
            + jnp.where((r8 == 0) & (l8 == 2), fn, 0.0))
    out_ref[0] = vals


def _round_bf16(x):
    u = jax.lax.bitcast_convert_type(x, jnp.uint32)
    u = (u + jnp.uint32(0x7FFF) + ((u >> 16) & jnp.uint32(1))) & jnp.uint32(0xFFFF0000)
    return jax.lax.bitcast_convert_type(u, jnp.float32)


def _prep(pts):
    conf = pts[:, :, 0, :]
    zc = pts[:, :, 1, :]
    xc = pts[:, :, 2, :]
    key = jnp.where(conf > _THRESH, conf, -jnp.inf)
    order = jnp.argsort(-key, axis=-1)
    cs = jnp.take_along_axis(conf, order, axis=-1)
    zs = jnp.take_along_axis(zc, order, axis=-1)
    xs = jnp.take_along_axis(xc, order, axis=-1)
    S = pts.shape[0] * pts.shape[1]
    sh = (S, _ROWS, _COLS)
    out = [cs.reshape(sh), zs.reshape(sh), xs.reshape(sh)]
    out += [_round_bf16(a) for a in out]
    return out


def kernel(predictions, targets):
    if predictions.ndim == 4:
        predictions = predictions[None]
        targets = targets[None]
    b = predictions.shape[0]
    ind = _grid_ind()
    scale = _chan_scale()
    preds = _organize(predictions, ind, scale)
    targs = _organize(targets, ind, scale)
    P6 = _prep(preds)
    T6 = _prep(targs)
    S = b * 2
    r2_pair = jnp.asarray([_RAD["O"] ** 2, _RAD["H"] ** 2], dtype=jnp.float32)
    r2 = jnp.tile(r2_pair, b)
    r2b = jnp.broadcast_to(r2[:, None, None], (S, 8, 128))

    vspec = pl.BlockSpec((1, _ROWS, _COLS), lambda i: (i, 0, 0))
    out = pl.pallas_call(
        _slice_kernel,
        grid=(S,),
        in_specs=[pl.BlockSpec((1, 8, 128), lambda i: (i, 0, 0))] + [vspec] * 12,
        out_specs=pl.BlockSpec((1, 8, 128), lambda i: (i, 0, 0)),
        out_shape=jax.ShapeDtypeStruct((S, 8, 128), jnp.float32),
        scratch_shapes=[pltpu.VMEM((_ROWS, _COLS), jnp.float32)] * 3,
    )(r2b, *P6, *T6)
    return out[:, 0, :3].reshape(b, 2, 3)
